# baseline (speedup 1.0000x reference)
import jax
import jax.numpy as jnp
import numpy as np
from jax.experimental import pallas as pl
from jax.experimental.pallas import tpu as pltpu
from jax.sharding import Mesh, PartitionSpec as P

QK_DIM = 32
TOPK = 16
N = 2048
BQ = 256


def _topk_route_kernel(q_ref, k_ref, out_ref):
    scale = QK_DIM ** (-0.5)
    q = q_ref[0] * jnp.float32(scale)
    k = k_ref[0]
    logits = jax.lax.dot_general(
        q, k, (((1,), (1,)), ((), ())),
        preferred_element_type=jnp.float32)

    col = jax.lax.broadcasted_iota(jnp.int32, logits.shape, 1)
    neg = jnp.float32(-jnp.inf)
    outs = []
    for _ in range(TOPK):
        idx = jnp.argmax(logits, axis=1, keepdims=True)
        outs.append(idx)
        logits = jnp.where(col == idx, neg, logits)

    out_ref[0] = jnp.concatenate(outs, axis=1)


def _topk_call(query, key):
    batch = query.shape[0]
    grid = (batch, N // BQ)
    return pl.pallas_call(
        _topk_route_kernel,
        grid=grid,
        in_specs=[
            pl.BlockSpec((1, BQ, QK_DIM), lambda b, i: (b, i, 0)),
            pl.BlockSpec((1, N, QK_DIM), lambda b, i: (b, 0, 0)),
        ],
        out_specs=pl.BlockSpec((1, BQ, TOPK), lambda b, i: (b, i, 0)),
        out_shape=jax.ShapeDtypeStruct((batch, N, TOPK), jnp.int32),
        compiler_params=pltpu.CompilerParams(
            dimension_semantics=("parallel", "parallel")),
    )(query, key)


def kernel(query, key):
    batch = query.shape[0]
    devs = jax.devices()
    ndev = 2 if (len(devs) >= 2 and batch % 2 == 0) else 1
    if ndev == 1:
        return _topk_call(query, key)
    mesh = Mesh(np.array(devs[:ndev]), ("b",))
    f = jax.shard_map(_topk_call, mesh=mesh, check_vma=False,
                      in_specs=(P("b"), P("b")), out_specs=P("b"))
    return f(query, key)


# speedup vs baseline: 1.5899x; 1.5899x over previous
import jax
import jax.numpy as jnp
from jax.experimental import pallas as pl
from jax.experimental.pallas import tpu as pltpu

QK_DIM = 32
TOPK = 16
N = 2048
BQ = 256


def _topk_route_kernel(q_ref, k_ref, out_ref):
    scale = QK_DIM ** (-0.5)
    q = q_ref[0] * jnp.float32(scale)
    k = k_ref[0]
    logits = jax.lax.dot_general(
        q, k, (((1,), (1,)), ((), ())),
        preferred_element_type=jnp.float32)

    col = jax.lax.broadcasted_iota(jnp.int32, logits.shape, 1)
    neg = jnp.float32(-jnp.inf)
    outs = []
    for r in range(TOPK):
        idx = jnp.argmax(logits, axis=1, keepdims=True)
        outs.append(idx)
        if r < TOPK - 1:
            logits = jnp.where(col == idx, neg, logits)

    out_ref[0] = jnp.concatenate(outs, axis=1)


def _topk_call(query, key):
    batch = query.shape[0]
    grid = (batch, N // BQ)
    return pl.pallas_call(
        _topk_route_kernel,
        grid=grid,
        in_specs=[
            pl.BlockSpec((1, BQ, QK_DIM), lambda b, i: (b, i, 0)),
            pl.BlockSpec((1, N, QK_DIM), lambda b, i: (b, 0, 0)),
        ],
        out_specs=pl.BlockSpec((1, BQ, TOPK), lambda b, i: (b, i, 0)),
        out_shape=jax.ShapeDtypeStruct((batch, N, TOPK), jnp.int32),
        compiler_params=pltpu.CompilerParams(
            dimension_semantics=("parallel", "parallel")),
    )(query, key)


def kernel(query, key):
    return _topk_call(query, key)
